# Initial kernel scaffold; baseline (speedup 1.0000x reference)
import jax
import jax.numpy as jnp
from jax.experimental import pallas as pl


def kernel(x, pos_table, gamma, beta):
    raise NotImplementedError("write your pallas kernel here")


# speedup vs baseline: 1.8866x; 1.8866x over previous
import jax
import jax.numpy as jnp
from jax.experimental import pallas as pl
from jax.experimental.pallas import tpu as pltpu

_SEQ_TILE = 512


def _ln_kernel(x_ref, pos_ref, gamma_ref, beta_ref, out_ref):
    e = x_ref[0] + pos_ref[...]
    mean = jnp.mean(e, axis=1, keepdims=True)
    c = e - mean
    var = jnp.mean(c * c, axis=1, keepdims=True)
    inv = jax.lax.rsqrt(var + 1e-12)
    out_ref[0] = c * inv * gamma_ref[...] + beta_ref[...]


def kernel(x, pos_table, gamma, beta):
    B, S, D = x.shape
    ts = _SEQ_TILE
    gamma2 = gamma.reshape(1, D)
    beta2 = beta.reshape(1, D)
    grid = (B, S // ts)
    return pl.pallas_call(
        _ln_kernel,
        grid=grid,
        in_specs=[
            pl.BlockSpec((1, ts, D), lambda b, s: (b, s, 0)),
            pl.BlockSpec((ts, D), lambda b, s: (s, 0)),
            pl.BlockSpec((1, D), lambda b, s: (0, 0)),
            pl.BlockSpec((1, D), lambda b, s: (0, 0)),
        ],
        out_specs=pl.BlockSpec((1, ts, D), lambda b, s: (b, s, 0)),
        out_shape=jax.ShapeDtypeStruct((B, S, D), x.dtype),
        compiler_params=pltpu.CompilerParams(
            dimension_semantics=("parallel", "parallel"),
        ),
    )(x, pos_table, gamma2, beta2)


# speedup vs baseline: 1.9948x; 1.0573x over previous
import jax
import jax.numpy as jnp
from jax.experimental import pallas as pl
from jax.experimental.pallas import tpu as pltpu

_SEQ_TILE = 512


def _ln_kernel(x_ref, pos_ref, gamma_ref, beta_ref, out_ref):
    e = x_ref[0] + pos_ref[...]
    mean = jnp.mean(e, axis=1, keepdims=True)
    c = e - mean
    var = jnp.mean(c * c, axis=1, keepdims=True)
    inv = jax.lax.rsqrt(var + 1e-12)
    out_ref[0] = c * inv * gamma_ref[...] + beta_ref[...]


def kernel(x, pos_table, gamma, beta):
    B, S, D = x.shape
    ts = _SEQ_TILE
    gamma2 = gamma.reshape(1, D)
    beta2 = beta.reshape(1, D)
    grid = (S // ts, B)
    return pl.pallas_call(
        _ln_kernel,
        grid=grid,
        in_specs=[
            pl.BlockSpec((1, ts, D), lambda s, b: (b, s, 0)),
            pl.BlockSpec((ts, D), lambda s, b: (s, 0)),
            pl.BlockSpec((1, D), lambda s, b: (0, 0)),
            pl.BlockSpec((1, D), lambda s, b: (0, 0)),
        ],
        out_specs=pl.BlockSpec((1, ts, D), lambda s, b: (b, s, 0)),
        out_shape=jax.ShapeDtypeStruct((B, S, D), x.dtype),
        compiler_params=pltpu.CompilerParams(
            dimension_semantics=("parallel", "parallel"),
        ),
    )(x, pos_table, gamma2, beta2)


# speedup vs baseline: 2.3528x; 1.1795x over previous
import jax
import jax.numpy as jnp
from jax.experimental import pallas as pl
from jax.experimental.pallas import tpu as pltpu

_SEQ_TILE = 1024


def _ln_kernel(x_ref, pos_ref, gamma_ref, beta_ref, out_ref):
    e = x_ref[0] + pos_ref[...]
    mean = jnp.mean(e, axis=1, keepdims=True)
    c = e - mean
    var = jnp.mean(c * c, axis=1, keepdims=True)
    inv = jax.lax.rsqrt(var + 1e-12)
    out_ref[0] = c * inv * gamma_ref[...] + beta_ref[...]


def kernel(x, pos_table, gamma, beta):
    B, S, D = x.shape
    ts = _SEQ_TILE
    gamma2 = gamma.reshape(1, D)
    beta2 = beta.reshape(1, D)
    grid = (S // ts, B)
    return pl.pallas_call(
        _ln_kernel,
        grid=grid,
        in_specs=[
            pl.BlockSpec((1, ts, D), lambda s, b: (b, s, 0)),
            pl.BlockSpec((ts, D), lambda s, b: (s, 0)),
            pl.BlockSpec((1, D), lambda s, b: (0, 0)),
            pl.BlockSpec((1, D), lambda s, b: (0, 0)),
        ],
        out_specs=pl.BlockSpec((1, ts, D), lambda s, b: (b, s, 0)),
        out_shape=jax.ShapeDtypeStruct((B, S, D), x.dtype),
        compiler_params=pltpu.CompilerParams(
            dimension_semantics=("parallel", "parallel"),
        ),
    )(x, pos_table, gamma2, beta2)


# speedup vs baseline: 2.5492x; 1.0835x over previous
import jax
import jax.numpy as jnp
from jax.experimental import pallas as pl
from jax.experimental.pallas import tpu as pltpu

_SEQ_TILE = 2048


def _ln_kernel(x_ref, pos_ref, gamma_ref, beta_ref, out_ref):
    e = x_ref[0] + pos_ref[...]
    mean = jnp.mean(e, axis=1, keepdims=True)
    c = e - mean
    var = jnp.mean(c * c, axis=1, keepdims=True)
    inv = jax.lax.rsqrt(var + 1e-12)
    out_ref[0] = c * inv * gamma_ref[...] + beta_ref[...]


def kernel(x, pos_table, gamma, beta):
    B, S, D = x.shape
    ts = _SEQ_TILE
    gamma2 = gamma.reshape(1, D)
    beta2 = beta.reshape(1, D)
    grid = (S // ts, B)
    return pl.pallas_call(
        _ln_kernel,
        grid=grid,
        in_specs=[
            pl.BlockSpec((1, ts, D), lambda s, b: (b, s, 0)),
            pl.BlockSpec((ts, D), lambda s, b: (s, 0)),
            pl.BlockSpec((1, D), lambda s, b: (0, 0)),
            pl.BlockSpec((1, D), lambda s, b: (0, 0)),
        ],
        out_specs=pl.BlockSpec((1, ts, D), lambda s, b: (b, s, 0)),
        out_shape=jax.ShapeDtypeStruct((B, S, D), x.dtype),
        compiler_params=pltpu.CompilerParams(
            dimension_semantics=("parallel", "parallel"),
        ),
    )(x, pos_table, gamma2, beta2)
